# Initial kernel scaffold; baseline (speedup 1.0000x reference)
import functools

import jax
import jax.numpy as jnp
from jax import lax
from jax.experimental import pallas as pl
from jax.experimental.pallas import tpu as pltpu
from jax.experimental.pallas import tpu_sc as plsc

NC = 2
NS = 16
NW = NC * NS
LANES = 16


def _ymat_body(x_ref, w_ref, b_ref, y_ref):
    y = lax.dot_general(
        x_ref[...], w_ref[0],
        dimension_numbers=(((1,), (1,)), ((), ())),
        preferred_element_type=jnp.float32,
    )
    y_ref[0] = y + b_ref[0][None, :]


def _compute_y(x, weight, bias, bn):
    n, d_in = x.shape
    t, d_out, _ = weight.shape
    grid = (t, n // bn)
    return pl.pallas_call(
        _ymat_body,
        grid=grid,
        in_specs=[
            pl.BlockSpec((bn, d_in), lambda ti, i: (i, 0)),
            pl.BlockSpec((1, d_out, d_in), lambda ti, i: (ti, 0, 0)),
            pl.BlockSpec((1, d_out), lambda ti, i: (ti, 0)),
        ],
        out_specs=pl.BlockSpec((1, bn, d_out), lambda ti, i: (ti, i, 0)),
        out_shape=jax.ShapeDtypeStruct((t, n, d_out), jnp.float32),
    )(x, weight, bias)


def _add_body(a_ref, b_ref, o_ref):
    o_ref[...] = a_ref[...] + b_ref[...]


def _combine_partials(partial, n, d, bn):
    nb = n // bn
    return pl.pallas_call(
        _add_body,
        grid=(nb,),
        in_specs=[
            pl.BlockSpec((bn, d), lambda i: (i, 0)),
            pl.BlockSpec((bn, d), lambda i: (i + nb, 0)),
        ],
        out_specs=pl.BlockSpec((bn, d), lambda i: (i, 0)),
        out_shape=jax.ShapeDtypeStruct((n, d), jnp.float32),
    )(partial, partial)


def _sc_scatter(y2, src, tgt, etype, n_nodes, d, ch):
    e = src.shape[0]
    epw = e // NW
    nch = epw // ch
    rpt = n_nodes // NS
    zrows = 125
    mesh = plsc.VectorSubcoreMesh(
        core_axis_name="c", subcore_axis_name="s", num_cores=NC, num_subcores=NS)

    @functools.partial(
        pl.kernel,
        mesh=mesh,
        out_type=jax.ShapeDtypeStruct((NC * n_nodes, d), jnp.float32),
        scratch_types=[
            pltpu.VMEM((epw,), jnp.int32),
            pltpu.VMEM((epw,), jnp.int32),
            pltpu.VMEM((epw,), jnp.int32),
            pltpu.VMEM((nch, ch), jnp.int32),
            pltpu.VMEM((nch, ch), jnp.int32),
            pltpu.VMEM((ch, d), jnp.float32),
            pltpu.VMEM((zrows, d), jnp.float32),
            pltpu.VMEM_SHARED((n_nodes, d), jnp.float32),
            pltpu.SemaphoreType.DMA,
        ],
    )
    def k(y_h, src_h, tgt_h, typ_h, out_h, esrc, etyp, etgt, gidx, tidx,
          rows, zbuf, acc, sem):
        c = lax.axis_index("c")
        s = lax.axis_index("s")
        wid = c * NS + s
        base = wid * epw

        pltpu.sync_copy(src_h.at[pl.ds(base, epw)], esrc)
        pltpu.sync_copy(typ_h.at[pl.ds(base, epw)], etyp)
        pltpu.sync_copy(tgt_h.at[pl.ds(base, epw)], etgt)

        def idx_body(j, carry):
            for v in range(ch // LANES):
                off = j * ch + v * LANES
                sv = esrc[pl.ds(off, LANES)]
                tv = etyp[pl.ds(off, LANES)]
                gidx[j, pl.ds(v * LANES, LANES)] = tv * n_nodes + sv
                tidx[j, pl.ds(v * LANES, LANES)] = etgt[pl.ds(off, LANES)]
            return carry
        lax.fori_loop(0, nch, idx_body, 0)

        zero = jnp.zeros((LANES,), jnp.float32)
        def zrow_body(i, carry):
            for v in range(d // LANES):
                zbuf[i, pl.ds(v * LANES, LANES)] = zero
            return carry
        lax.fori_loop(0, zrows, zrow_body, 0)
        for kk in range(rpt // zrows):
            pltpu.sync_copy(zbuf, acc.at[pl.ds(s * rpt + kk * zrows, zrows)])
        plsc.subcore_barrier()

        def main_body(j, carry):
            pltpu.async_copy(y_h.at[gidx.at[j]], rows, sem).wait()
            pltpu.sync_copy(rows, acc.at[tidx.at[j]], add=True)
            return carry
        lax.fori_loop(0, nch, main_body, 0)

        plsc.subcore_barrier()
        pltpu.sync_copy(acc.at[pl.ds(s * rpt, rpt)],
                        out_h.at[pl.ds(c * n_nodes + s * rpt, rpt)])

    return k(y2, src, tgt, etype)


def kernel(x, edge_index, edge_type, weight, bias):
    n, d_in = x.shape
    t, d_out, _ = weight.shape
    y = _compute_y(x, weight, bias, bn=1000)
    y2 = y.reshape(t * n, d_out)
    partial = _sc_scatter(y2, edge_index[0], edge_index[1], edge_type,
                          n_nodes=n, d=d_out, ch=80)
    return _combine_partials(partial, n, d_out, bn=1000)


# speedup vs baseline: 14.3347x; 14.3347x over previous
import functools

import jax
import jax.numpy as jnp
from jax import lax
from jax.experimental import pallas as pl
from jax.experimental.pallas import tpu as pltpu
from jax.experimental.pallas import tpu_sc as plsc

NC = 2
NS = 16
NW = NC * NS
LANES = 16


def _ymat_body(x_ref, w_ref, b_ref, y_ref):
    y = lax.dot_general(
        x_ref[...], w_ref[0],
        dimension_numbers=(((1,), (1,)), ((), ())),
        preferred_element_type=jnp.float32,
    )
    y_ref[0] = y + b_ref[0]


def _compute_y(x, weight, bias, bn):
    n, d_in = x.shape
    t, d_out, _ = weight.shape
    grid = (t, n // bn)
    return pl.pallas_call(
        _ymat_body,
        grid=grid,
        in_specs=[
            pl.BlockSpec((bn, d_in), lambda ti, i: (i, 0)),
            pl.BlockSpec((1, d_out, d_in), lambda ti, i: (ti, 0, 0)),
            pl.BlockSpec((1, 1, d_out), lambda ti, i: (ti, 0, 0)),
        ],
        out_specs=pl.BlockSpec((1, bn, d_out), lambda ti, i: (ti, i, 0)),
        out_shape=jax.ShapeDtypeStruct((t, n, d_out), jnp.float32),
    )(x, weight, bias.reshape(t, 1, d_out))


def _add_body(a_ref, b_ref, o_ref):
    o_ref[...] = a_ref[...] + b_ref[...]


def _combine_partials(partial, n, d, bn):
    nb = n // bn
    return pl.pallas_call(
        _add_body,
        grid=(nb,),
        in_specs=[
            pl.BlockSpec((bn, d), lambda i: (i, 0)),
            pl.BlockSpec((bn, d), lambda i: (i + nb, 0)),
        ],
        out_specs=pl.BlockSpec((bn, d), lambda i: (i, 0)),
        out_shape=jax.ShapeDtypeStruct((n, d), jnp.float32),
    )(partial, partial)


def _sc_scatter(y2, src, tgt, etype, n_nodes, d, ch, eb):
    e = src.shape[0]
    epw = e // NW
    nblk = epw // eb
    nchb = eb // ch
    full = (n_nodes // NS) & ~7
    rem = n_nodes - NS * full
    nz, ztail = full // ch, full % ch
    assert full % 8 == 0 and rem % 8 == 0 and ztail % 8 == 0 and rem <= ch
    assert e % NW == 0 and epw % eb == 0 and eb % ch == 0 and ch % LANES == 0
    mesh = plsc.VectorSubcoreMesh(
        core_axis_name="c", subcore_axis_name="s", num_cores=NC, num_subcores=NS)

    @functools.partial(
        pl.kernel,
        mesh=mesh,
        out_type=jax.ShapeDtypeStruct((NC * n_nodes, d), jnp.float32),
        scratch_types=[
            pltpu.VMEM((eb,), jnp.int32),
            pltpu.VMEM((eb,), jnp.int32),
            pltpu.VMEM((eb,), jnp.int32),
            pltpu.VMEM((nchb, ch), jnp.int32),
            pltpu.VMEM((nchb, ch), jnp.int32),
            pltpu.VMEM((ch, d), jnp.float32),
            pltpu.VMEM_SHARED((n_nodes, d), jnp.float32),
            pltpu.SemaphoreType.DMA,
        ],
    )
    def k(y_h, src_h, tgt_h, typ_h, out_h, bsrc, btyp, btgt, gidx, tidx,
          rows, acc, sem):
        c = lax.axis_index("c")
        s = lax.axis_index("s")
        wid = c * NS + s
        base = wid * epw

        zero = jnp.zeros((LANES,), jnp.float32)
        def zrow_body(i, carry):
            for v in range(d // LANES):
                rows[i, pl.ds(v * LANES, LANES)] = zero
            return carry
        lax.fori_loop(0, ch, zrow_body, 0)
        a0 = s * full
        for kk in range(nz):
            pltpu.sync_copy(rows, acc.at[pl.ds(a0 + kk * ch, ch)])
        if ztail:
            pltpu.sync_copy(rows.at[pl.ds(0, ztail)],
                            acc.at[pl.ds(a0 + nz * ch, ztail)])
        @pl.when(s == NS - 1)
        def _zero_tail():
            pltpu.sync_copy(rows.at[pl.ds(0, rem)],
                            acc.at[pl.ds(NS * full, rem)])
        plsc.subcore_barrier()

        def blk_body(blk, carry):
            bbase = base + blk * eb
            pltpu.sync_copy(src_h.at[pl.ds(bbase, eb)], bsrc)
            pltpu.sync_copy(typ_h.at[pl.ds(bbase, eb)], btyp)
            pltpu.sync_copy(tgt_h.at[pl.ds(bbase, eb)], btgt)

            def idx_body(j, carry2):
                for v in range(ch // LANES):
                    off = j * ch + v * LANES
                    sv = bsrc[pl.ds(off, LANES)]
                    tv = btyp[pl.ds(off, LANES)]
                    gidx[j, pl.ds(v * LANES, LANES)] = tv * n_nodes + sv
                    tidx[j, pl.ds(v * LANES, LANES)] = btgt[pl.ds(off, LANES)]
                return carry2
            lax.fori_loop(0, nchb, idx_body, 0)

            def main_body(j, carry2):
                pltpu.async_copy(y_h.at[gidx.at[j]], rows, sem).wait()
                pltpu.sync_copy(rows, acc.at[tidx.at[j]], add=True)
                return carry2
            lax.fori_loop(0, nchb, main_body, 0)
            return carry
        lax.fori_loop(0, nblk, blk_body, 0)

        plsc.subcore_barrier()
        pltpu.sync_copy(acc.at[pl.ds(a0, full)],
                        out_h.at[pl.ds(c * n_nodes + a0, full)])
        @pl.when(s == NS - 1)
        def _write_tail():
            pltpu.sync_copy(acc.at[pl.ds(NS * full, rem)],
                            out_h.at[pl.ds(c * n_nodes + NS * full, rem)])

    return k(y2, src, tgt, etype)


def kernel(x, edge_index, edge_type, weight, bias):
    n, d_in = x.shape
    t, d_out, _ = weight.shape
    y = _compute_y(x, weight, bias, bn=1000)
    y2 = y.reshape(t * n, d_out)
    partial = _sc_scatter(y2, edge_index[0], edge_index[1], edge_type,
                          n_nodes=n, d=d_out, ch=80, eb=2000)
    return _combine_partials(partial, n, d_out, bn=1000)


# speedup vs baseline: 22.5055x; 1.5700x over previous
import functools

import jax
import jax.numpy as jnp
from jax import lax
from jax.experimental import pallas as pl
from jax.experimental.pallas import tpu as pltpu
from jax.experimental.pallas import tpu_sc as plsc

NC = 2
NS = 16
NW = NC * NS
LANES = 16


def _ymat_body(x_ref, w_ref, b_ref, y_ref):
    y = lax.dot_general(
        x_ref[...], w_ref[0],
        dimension_numbers=(((1,), (1,)), ((), ())),
        preferred_element_type=jnp.float32,
    )
    y_ref[0] = y + b_ref[0]


def _compute_y(x, weight, bias, bn):
    n, d_in = x.shape
    t, d_out, _ = weight.shape
    grid = (t, n // bn)
    return pl.pallas_call(
        _ymat_body,
        grid=grid,
        in_specs=[
            pl.BlockSpec((bn, d_in), lambda ti, i: (i, 0)),
            pl.BlockSpec((1, d_out, d_in), lambda ti, i: (ti, 0, 0)),
            pl.BlockSpec((1, 1, d_out), lambda ti, i: (ti, 0, 0)),
        ],
        out_specs=pl.BlockSpec((1, bn, d_out), lambda ti, i: (ti, i, 0)),
        out_shape=jax.ShapeDtypeStruct((t, n, d_out), jnp.float32),
    )(x, weight, bias.reshape(t, 1, d_out))


def _pidx_call(edge_index, edge_type, n_nodes):
    e = edge_type.shape[0]
    rows, cols = e // 1280, 1280
    ei3 = edge_index.reshape(2, rows, cols)
    et2 = edge_type.reshape(rows, cols)

    def body(src_ref, tgt_ref, et_ref, o_ref):
        o_ref[...] = ((et_ref[...] * n_nodes + src_ref[0]) * 16384
                      + tgt_ref[0])

    out = pl.pallas_call(
        body,
        grid=(1,),
        in_specs=[
            pl.BlockSpec((1, rows, cols), lambda i: (0, 0, 0)),
            pl.BlockSpec((1, rows, cols), lambda i: (1, 0, 0)),
            pl.BlockSpec((rows, cols), lambda i: (0, 0)),
        ],
        out_specs=pl.BlockSpec((rows, cols), lambda i: (0, 0)),
        out_shape=jax.ShapeDtypeStruct((rows, cols), jnp.int32),
    )(ei3, ei3, et2)
    return out.reshape(e)


def _add_body(a_ref, b_ref, o_ref):
    o_ref[...] = a_ref[...] + b_ref[...]


def _combine_partials(partial, n, d, bn):
    nb = n // bn
    return pl.pallas_call(
        _add_body,
        grid=(nb,),
        in_specs=[
            pl.BlockSpec((bn, d), lambda i: (i, 0)),
            pl.BlockSpec((bn, d), lambda i: (i + nb, 0)),
        ],
        out_specs=pl.BlockSpec((bn, d), lambda i: (i, 0)),
        out_shape=jax.ShapeDtypeStruct((n, d), jnp.float32),
    )(partial, partial)


def _sc_scatter(y2, pidx, n_nodes, d, ch):
    e = pidx.shape[0]
    epw = e // NW
    nch = epw // ch
    npairs = (nch - 1) // 2
    full = (n_nodes // NS) & ~7
    rem = n_nodes - NS * full
    nz, ztail = full // ch, full % ch
    assert full % 8 == 0 and rem % 8 == 0 and ztail % 8 == 0 and rem <= ch
    assert e % NW == 0 and epw % ch == 0 and ch % LANES == 0 and nch % 2 == 1
    mesh = plsc.VectorSubcoreMesh(
        core_axis_name="c", subcore_axis_name="s", num_cores=NC, num_subcores=NS)

    @functools.partial(
        pl.kernel,
        mesh=mesh,
        out_type=jax.ShapeDtypeStruct((NC * n_nodes, d), jnp.float32),
        scratch_types=[
            pltpu.VMEM((epw,), jnp.int32),
            pltpu.VMEM((ch,), jnp.int32),
            pltpu.VMEM((ch,), jnp.int32),
            pltpu.VMEM((ch,), jnp.int32),
            pltpu.VMEM((ch,), jnp.int32),
            pltpu.VMEM((ch, d), jnp.float32),
            pltpu.VMEM((ch, d), jnp.float32),
            pltpu.VMEM_SHARED((n_nodes, d), jnp.float32),
            pltpu.SemaphoreType.DMA,
            pltpu.SemaphoreType.DMA,
            pltpu.SemaphoreType.DMA,
        ],
    )
    def k(y_h, p_h, out_h, pvm, g0, t0, g1, t1, rows0, rows1, acc,
          sem0, sem1, semp):
        c = lax.axis_index("c")
        s = lax.axis_index("s")
        base = (c * NS + s) * epw
        pdma = pltpu.async_copy(p_h.at[pl.ds(base, epw)], pvm, semp)

        zero = jnp.zeros((LANES,), jnp.float32)
        def zrow_body(i, carry):
            for v in range(d // LANES):
                rows0[i, pl.ds(v * LANES, LANES)] = zero
            return carry
        lax.fori_loop(0, ch, zrow_body, 0)
        a0 = s * full
        for kk in range(nz):
            pltpu.sync_copy(rows0, acc.at[pl.ds(a0 + kk * ch, ch)])
        if ztail:
            pltpu.sync_copy(rows0.at[pl.ds(0, ztail)],
                            acc.at[pl.ds(a0 + nz * ch, ztail)])
        @pl.when(s == NS - 1)
        def _zero_tail():
            pltpu.sync_copy(rows0.at[pl.ds(0, rem)],
                            acc.at[pl.ds(NS * full, rem)])
        pdma.wait()
        plsc.subcore_barrier()

        def decode(j, gt, tt):
            for v in range(ch // LANES):
                pv = pvm[pl.ds(j * ch + v * LANES, LANES)]
                gt[pl.ds(v * LANES, LANES)] = pv >> 14
                tt[pl.ds(v * LANES, LANES)] = pv & 16383

        decode(0, g0, t0)
        cp0 = pltpu.async_copy(y_h.at[g0], rows0, sem0)
        decode(1, g1, t1)
        pltpu.async_copy(y_h.at[g1], rows1, sem1)
        cp0.wait()
        pltpu.sync_copy(rows0, acc.at[t0], add=True)
        decode(2, g0, t0)
        pltpu.async_copy(y_h.at[g0], rows0, sem0)

        def pair(i, carry):
            a = 2 * i + 1
            pltpu.make_async_copy(y_h.at[g1], rows1, sem1).wait()
            pltpu.sync_copy(rows1, acc.at[t1], add=True)
            @pl.when(a + 2 < nch)
            def _():
                decode(a + 2, g1, t1)
                pltpu.async_copy(y_h.at[g1], rows1, sem1)
            pltpu.make_async_copy(y_h.at[g0], rows0, sem0).wait()
            pltpu.sync_copy(rows0, acc.at[t0], add=True)
            @pl.when(a + 3 < nch)
            def _():
                decode(a + 3, g0, t0)
                pltpu.async_copy(y_h.at[g0], rows0, sem0)
            return carry
        lax.fori_loop(0, npairs, pair, 0)

        plsc.subcore_barrier()
        pltpu.sync_copy(acc.at[pl.ds(a0, full)],
                        out_h.at[pl.ds(c * n_nodes + a0, full)])
        @pl.when(s == NS - 1)
        def _write_tail():
            pltpu.sync_copy(acc.at[pl.ds(NS * full, rem)],
                            out_h.at[pl.ds(c * n_nodes + NS * full, rem)])

    return k(y2, pidx)


def kernel(x, edge_index, edge_type, weight, bias):
    n, d_in = x.shape
    t, d_out, _ = weight.shape
    y = _compute_y(x, weight, bias, bn=1000)
    y2 = y.reshape(t * n, d_out)
    pidx = _pidx_call(edge_index, edge_type, n_nodes=n)
    partial = _sc_scatter(y2, pidx, n_nodes=n, d=d_out, ch=80)
    return _combine_partials(partial, n, d_out, bn=1000)


# speedup vs baseline: 25.3063x; 1.1245x over previous
import functools

import jax
import jax.numpy as jnp
from jax import lax
from jax.experimental import pallas as pl
from jax.experimental.pallas import tpu as pltpu
from jax.experimental.pallas import tpu_sc as plsc

NC = 2
NS = 16
NW = NC * NS
LANES = 16


def _compute_y(x, weight, bias, bn):
    n, d_in = x.shape
    t, d_out, _ = weight.shape

    def body(x_ref, w_ref, b_ref, y_ref):
        for ti in range(t):
            y = lax.dot_general(
                x_ref[...], w_ref[ti],
                dimension_numbers=(((1,), (1,)), ((), ())),
                preferred_element_type=jnp.float32,
            )
            y_ref[ti] = y + b_ref[ti]

    return pl.pallas_call(
        body,
        grid=(n // bn,),
        in_specs=[
            pl.BlockSpec((bn, d_in), lambda i: (i, 0)),
            pl.BlockSpec((t, d_out, d_in), lambda i: (0, 0, 0)),
            pl.BlockSpec((t, 1, d_out), lambda i: (0, 0, 0)),
        ],
        out_specs=pl.BlockSpec((t, bn, d_out), lambda i: (0, i, 0)),
        out_shape=jax.ShapeDtypeStruct((t, n, d_out), jnp.float32),
    )(x, weight, bias.reshape(t, 1, d_out))


def _pidx_call(edge_index, edge_type, n_nodes):
    e = edge_type.shape[0]
    rows, cols = e // 1280, 1280
    ei3 = edge_index.reshape(2, rows, cols)
    et2 = edge_type.reshape(rows, cols)

    def body(src_ref, tgt_ref, et_ref, o_ref):
        o_ref[...] = ((et_ref[...] * n_nodes + src_ref[0]) * 16384
                      + tgt_ref[0])

    out = pl.pallas_call(
        body,
        grid=(1,),
        in_specs=[
            pl.BlockSpec((1, rows, cols), lambda i: (0, 0, 0)),
            pl.BlockSpec((1, rows, cols), lambda i: (1, 0, 0)),
            pl.BlockSpec((rows, cols), lambda i: (0, 0)),
        ],
        out_specs=pl.BlockSpec((rows, cols), lambda i: (0, 0)),
        out_shape=jax.ShapeDtypeStruct((rows, cols), jnp.int32),
    )(ei3, ei3, et2)
    return out.reshape(e)


def _add_body(a_ref, b_ref, o_ref):
    o_ref[...] = a_ref[...] + b_ref[...]


def _combine_partials(partial, n, d, bn):
    nb = n // bn
    return pl.pallas_call(
        _add_body,
        grid=(nb,),
        in_specs=[
            pl.BlockSpec((bn, d), lambda i: (i, 0)),
            pl.BlockSpec((bn, d), lambda i: (i + nb, 0)),
        ],
        out_specs=pl.BlockSpec((bn, d), lambda i: (i, 0)),
        out_shape=jax.ShapeDtypeStruct((n, d), jnp.float32),
    )(partial, partial)


def _sc_scatter(y2, pidx, n_nodes, d, ch):
    e = pidx.shape[0]
    epw = e // NW
    nch = epw // ch
    npairs = (nch - 1) // 2
    full = (n_nodes // NS) & ~7
    rem = n_nodes - NS * full
    nz, ztail = full // ch, full % ch
    assert full % 8 == 0 and rem % 8 == 0 and ztail % 8 == 0 and rem <= ch
    assert e % NW == 0 and epw % ch == 0 and ch % LANES == 0 and nch % 2 == 1
    mesh = plsc.VectorSubcoreMesh(
        core_axis_name="c", subcore_axis_name="s", num_cores=NC, num_subcores=NS)

    @functools.partial(
        pl.kernel,
        mesh=mesh,
        out_type=jax.ShapeDtypeStruct((NC * n_nodes, d), jnp.float32),
        scratch_types=[
            pltpu.VMEM((epw,), jnp.int32),
            pltpu.VMEM((ch,), jnp.int32),
            pltpu.VMEM((ch,), jnp.int32),
            pltpu.VMEM((ch,), jnp.int32),
            pltpu.VMEM((ch,), jnp.int32),
            pltpu.VMEM((ch, d), jnp.float32),
            pltpu.VMEM((ch, d), jnp.float32),
            pltpu.VMEM((40, d), jnp.float32),
            pltpu.VMEM_SHARED((n_nodes, d), jnp.float32),
            pltpu.SemaphoreType.DMA,
            pltpu.SemaphoreType.DMA,
            pltpu.SemaphoreType.DMA,
            pltpu.SemaphoreType.DMA,
            pltpu.SemaphoreType.DMA,
        ],
    )
    def k(y_h, p_h, out_h, pvm, g0, t0, g1, t1, rows0, rows1, zbuf, acc,
          sem0, sem1, ssc0, ssc1, semp):
        c = lax.axis_index("c")
        s = lax.axis_index("s")
        base = (c * NS + s) * epw
        pdma = pltpu.async_copy(p_h.at[pl.ds(base, epw)], pvm, semp)

        zero = jnp.zeros((LANES,), jnp.float32)
        def zrow_body(i, carry):
            for v in range(d // LANES):
                zbuf[i, pl.ds(v * LANES, LANES)] = zero
            return carry
        lax.fori_loop(0, 40, zrow_body, 0)

        def decode(j, gt, tt):
            for v in range(ch // LANES):
                pv = pvm[pl.ds(j * ch + v * LANES, LANES)]
                gt[pl.ds(v * LANES, LANES)] = pv >> 14
                tt[pl.ds(v * LANES, LANES)] = pv & 16383

        pdma.wait()
        decode(0, g0, t0)
        pltpu.async_copy(y_h.at[g0], rows0, sem0)
        decode(1, g1, t1)
        pltpu.async_copy(y_h.at[g1], rows1, sem1)

        a0 = s * full
        for kk in range(full // 40):
            pltpu.sync_copy(zbuf, acc.at[pl.ds(a0 + kk * 40, 40)])
        zt = full % 40
        if zt:
            pltpu.sync_copy(zbuf.at[pl.ds(0, zt)],
                            acc.at[pl.ds(a0 + (full // 40) * 40, zt)])
        @pl.when(s == NS - 1)
        def _zero_tail():
            pltpu.sync_copy(zbuf.at[pl.ds(0, rem)],
                            acc.at[pl.ds(NS * full, rem)])
        plsc.subcore_barrier()

        pltpu.make_async_copy(y_h.at[g0], rows0, sem0).wait()
        pltpu.async_copy(rows0, acc.at[t0], ssc0, add=True)

        def pair(i, carry):
            j1 = 2 * i + 1
            pltpu.make_async_copy(rows0, acc.at[t0], ssc0).wait()
            decode(j1 + 1, g0, t0)
            pltpu.async_copy(y_h.at[g0], rows0, sem0)
            pltpu.make_async_copy(y_h.at[g1], rows1, sem1).wait()
            pltpu.async_copy(rows1, acc.at[t1], ssc1, add=True)
            pltpu.make_async_copy(rows1, acc.at[t1], ssc1).wait()
            @pl.when(j1 + 2 < nch)
            def _():
                decode(j1 + 2, g1, t1)
                pltpu.async_copy(y_h.at[g1], rows1, sem1)
            pltpu.make_async_copy(y_h.at[g0], rows0, sem0).wait()
            pltpu.async_copy(rows0, acc.at[t0], ssc0, add=True)
            return carry
        lax.fori_loop(0, npairs, pair, 0)
        pltpu.make_async_copy(rows0, acc.at[t0], ssc0).wait()

        plsc.subcore_barrier()
        pltpu.sync_copy(acc.at[pl.ds(a0, full)],
                        out_h.at[pl.ds(c * n_nodes + a0, full)])
        @pl.when(s == NS - 1)
        def _write_tail():
            pltpu.sync_copy(acc.at[pl.ds(NS * full, rem)],
                            out_h.at[pl.ds(c * n_nodes + NS * full, rem)])

    return k(y2, pidx)


def kernel(x, edge_index, edge_type, weight, bias):
    n, d_in = x.shape
    t, d_out, _ = weight.shape
    y = _compute_y(x, weight, bias, bn=1000)
    y2 = y.reshape(t * n, d_out)
    pidx = _pidx_call(edge_index, edge_type, n_nodes=n)
    partial = _sc_scatter(y2, pidx, n_nodes=n, d=d_out, ch=80)
    return _combine_partials(partial, n, d_out, bn=1000)
